# baseline (speedup 1.0000x reference)
import functools

import jax
import jax.numpy as jnp
from jax import lax
from jax.experimental import pallas as pl
from jax.experimental.pallas import tpu as pltpu
from jax.experimental.pallas import tpu_sc as plsc

_NC = 2
_NS = 16


def _sc_gather(table, idx, b_per_w, nw_used):
    B = idx.shape[0]
    D = table.shape[1]
    mesh = plsc.VectorSubcoreMesh(core_axis_name="c", subcore_axis_name="s")

    @functools.partial(
        pl.kernel,
        mesh=mesh,
        compiler_params=pltpu.CompilerParams(use_tc_tiling_on_sc=False),
        out_type=jax.ShapeDtypeStruct((B, D), jnp.float32),
        scratch_types=[
            pltpu.VMEM((b_per_w,), jnp.int32),
            pltpu.VMEM((b_per_w, D), jnp.float32),
            pltpu.SemaphoreType.DMA,
        ],
    )
    def k(table_hbm, idx_hbm, out_hbm, idx_v, rows_v, sem):
        wid = lax.axis_index("s") * _NC + lax.axis_index("c")

        @pl.when(wid < nw_used)
        def _():
            base = wid * b_per_w
            pltpu.sync_copy(idx_hbm.at[pl.ds(base, b_per_w)], idx_v)
            pltpu.async_copy(table_hbm.at[idx_v], rows_v, sem).wait()
            pltpu.sync_copy(rows_v, out_hbm.at[pl.ds(base, b_per_w)])

    return k(table, idx)


def _tc_body(n_scales, vec2_ref, wa_ref, wb_ref, par_ref, sid_ref, fid_ref,
             tbl_ref, out_ref):
    br = out_ref.shape[0]
    x2 = vec2_ref[...]
    tok_a = lax.dot_general(
        x2, wa_ref[...], (((1,), (0,)), ((), ())),
        preferred_element_type=jnp.float32,
    )
    tok_b = lax.dot_general(
        x2, wb_ref[...], (((1,), (0,)), ((), ())),
        preferred_element_type=jnp.float32,
    )
    par = par_ref[0]
    tok = tok_a + par * (tok_b - tok_a)
    s = jnp.minimum(sid_ref[0], n_scales - 1)
    f = fid_ref[0] + 16
    iota = lax.broadcasted_iota(jnp.int32, (32, br), 0)
    oh = ((iota == s) | (iota == f)).astype(jnp.float32)
    emb = lax.dot_general(
        oh, tbl_ref[...], (((0,), (0,)), ((), ())),
        preferred_element_type=jnp.float32,
    )
    out_ref[...] = tok + emb


def kernel(tokens, scale_ids, frame_ids, codebook, W_proj, scale_table, frame_table):
    L = tokens.shape[0]
    D = codebook.shape[1]
    NE = W_proj.shape[0]
    n_scales = scale_table.shape[0]

    toks = tokens.astype(jnp.int32)
    cbp = codebook.reshape(codebook.shape[0] // 2, 2 * D)
    vec2 = _sc_gather(cbp, toks >> 1, 80, 28)

    Wa = jnp.concatenate([W_proj.T, jnp.zeros((D, NE), jnp.float32)], axis=0)
    Wb = jnp.concatenate([jnp.zeros((D, NE), jnp.float32), W_proj.T], axis=0)
    tbl = (
        jnp.zeros((32, NE), jnp.float32)
        .at[:n_scales].set(scale_table)
        .at[16:18].set(frame_table)
    )
    BR = 448
    grid = L // BR
    sids3 = scale_ids.astype(jnp.int32).reshape(grid, 1, BR)
    fids3 = frame_ids.astype(jnp.int32).reshape(grid, 1, BR)
    par3 = (toks & 1).astype(jnp.float32).reshape(grid, BR, 1)

    out = pl.pallas_call(
        functools.partial(_tc_body, n_scales),
        grid=(grid,),
        in_specs=[
            pl.BlockSpec((BR, 2 * D), lambda i: (i, 0)),
            pl.BlockSpec((2 * D, NE), lambda i: (0, 0)),
            pl.BlockSpec((2 * D, NE), lambda i: (0, 0)),
            pl.BlockSpec((1, BR, 1), lambda i: (i, 0, 0)),
            pl.BlockSpec((1, 1, BR), lambda i: (i, 0, 0)),
            pl.BlockSpec((1, 1, BR), lambda i: (i, 0, 0)),
            pl.BlockSpec((32, NE), lambda i: (0, 0)),
        ],
        out_specs=pl.BlockSpec((BR, NE), lambda i: (i, 0)),
        out_shape=jax.ShapeDtypeStruct((L, NE), jnp.float32),
    )(vec2, Wa, Wb, par3, sids3, fids3, tbl)
    return out


# speedup vs baseline: 1.1521x; 1.1521x over previous
import functools

import jax
import jax.numpy as jnp
from jax import lax
from jax.experimental import pallas as pl
from jax.experimental.pallas import tpu as pltpu
from jax.experimental.pallas import tpu_sc as plsc

_NC = 2
_NS = 16


def _sc_gather_packed(table, idx, b_per_w, nw_used):
    B = idx.shape[0]
    D = table.shape[1]
    half_rows = B // 2
    mesh = plsc.VectorSubcoreMesh(core_axis_name="c", subcore_axis_name="s")

    @functools.partial(
        pl.kernel,
        mesh=mesh,
        compiler_params=pltpu.CompilerParams(use_tc_tiling_on_sc=False),
        out_type=jax.ShapeDtypeStruct((half_rows, 2 * D), jnp.float32),
        scratch_types=[
            pltpu.VMEM((b_per_w,), jnp.int32),
            pltpu.VMEM((b_per_w, D), jnp.float32),
            pltpu.SemaphoreType.DMA,
        ],
    )
    def k(table_hbm, idx_hbm, out_hbm, idx_v, rows_v, sem):
        wid = lax.axis_index("s") * _NC + lax.axis_index("c")

        @pl.when(wid < nw_used)
        def _():
            base = wid * b_per_w
            half = wid // (nw_used // 2)
            rbase = base - half * half_rows
            lane = half * D
            pltpu.sync_copy(idx_hbm.at[pl.ds(base, b_per_w)], idx_v)
            pltpu.async_copy(table_hbm.at[idx_v], rows_v, sem).wait()
            pltpu.sync_copy(
                rows_v, out_hbm.at[pl.ds(rbase, b_per_w), pl.ds(lane, D)]
            )

    return k(table, idx)


def _tc_body(n_scales, vec2_ref, w2_ref, sid_ref, fid_ref, tbl_ref, out_ref):
    br = out_ref.shape[0]
    x2 = vec2_ref[...]
    w2 = w2_ref[0]
    tok = lax.dot_general(
        x2, w2, (((1,), (0,)), ((), ())), preferred_element_type=jnp.float32
    )
    s = jnp.minimum(sid_ref[0], n_scales - 1)
    f = fid_ref[0] + 16
    iota = lax.broadcasted_iota(jnp.int32, (32, br), 0)
    oh = ((iota == s) | (iota == f)).astype(jnp.float32)
    emb = lax.dot_general(
        oh, tbl_ref[...], (((0,), (0,)), ((), ())),
        preferred_element_type=jnp.float32,
    )
    out_ref[...] = tok + emb


def kernel(tokens, scale_ids, frame_ids, codebook, W_proj, scale_table, frame_table):
    L = tokens.shape[0]
    D = codebook.shape[1]
    NE = W_proj.shape[0]
    n_scales = scale_table.shape[0]

    vec2 = _sc_gather_packed(codebook, tokens.astype(jnp.int32), 80, 28)

    W2 = (
        jnp.zeros((2, 2 * D, NE), jnp.float32)
        .at[0, :D].set(W_proj.T)
        .at[1, D:].set(W_proj.T)
    )
    tbl = (
        jnp.zeros((32, NE), jnp.float32)
        .at[:n_scales].set(scale_table)
        .at[16:18].set(frame_table)
    )
    BR = 560
    grid = L // BR
    half_steps = grid // 2
    sids3 = scale_ids.astype(jnp.int32).reshape(grid, 1, BR)
    fids3 = frame_ids.astype(jnp.int32).reshape(grid, 1, BR)

    out = pl.pallas_call(
        functools.partial(_tc_body, n_scales),
        grid=(grid,),
        in_specs=[
            pl.BlockSpec((BR, 2 * D), lambda i: (i % half_steps, 0)),
            pl.BlockSpec((1, 2 * D, NE), lambda i: (i // half_steps, 0, 0)),
            pl.BlockSpec((1, 1, BR), lambda i: (i, 0, 0)),
            pl.BlockSpec((1, 1, BR), lambda i: (i, 0, 0)),
            pl.BlockSpec((32, NE), lambda i: (0, 0)),
        ],
        out_specs=pl.BlockSpec((BR, NE), lambda i: (i, 0)),
        out_shape=jax.ShapeDtypeStruct((L, NE), jnp.float32),
    )(vec2, W2, sids3, fids3, tbl)
    return out


# speedup vs baseline: 1.1844x; 1.0280x over previous
import functools

import jax
import jax.numpy as jnp
from jax import lax
from jax.experimental import pallas as pl
from jax.experimental.pallas import tpu as pltpu
from jax.experimental.pallas import tpu_sc as plsc

_NC = 2
_NS = 16


def _sc_gather_packed(table, idx, b_per_w, nw_used):
    B = idx.shape[0]
    D = table.shape[1]
    half_rows = B // 2
    mesh = plsc.VectorSubcoreMesh(core_axis_name="c", subcore_axis_name="s")

    @functools.partial(
        pl.kernel,
        mesh=mesh,
        compiler_params=pltpu.CompilerParams(use_tc_tiling_on_sc=False),
        out_type=jax.ShapeDtypeStruct((half_rows, 2 * D), jnp.float32),
        scratch_types=[
            pltpu.VMEM((b_per_w,), jnp.int32),
            pltpu.VMEM((b_per_w, D), jnp.float32),
            pltpu.SemaphoreType.DMA,
        ],
    )
    def k(table_hbm, idx_hbm, out_hbm, idx_v, rows_v, sem):
        wid = lax.axis_index("s") * _NC + lax.axis_index("c")

        @pl.when(wid < nw_used)
        def _():
            base = wid * b_per_w
            half = wid // (nw_used // 2)
            rbase = base - half * half_rows
            lane = half * D
            pltpu.sync_copy(idx_hbm.at[pl.ds(base, b_per_w)], idx_v)
            pltpu.async_copy(table_hbm.at[idx_v], rows_v, sem).wait()
            pltpu.sync_copy(
                rows_v, out_hbm.at[pl.ds(rbase, b_per_w), pl.ds(lane, D)]
            )

    return k(table, idx)


def _tc_body(n_scales, vec2_ref, w2_ref, sid_ref, fid_ref, tbl_ref, out_ref):
    br = out_ref.shape[0]
    x2 = vec2_ref[...]
    w2 = w2_ref[0]
    tok = lax.dot_general(
        x2, w2, (((1,), (0,)), ((), ())), preferred_element_type=jnp.float32
    )
    s = jnp.minimum(sid_ref[0], n_scales - 1)
    f = fid_ref[0] + 16
    iota = lax.broadcasted_iota(jnp.int32, (32, br), 0)
    oh = ((iota == s) | (iota == f)).astype(jnp.float32)
    emb = lax.dot_general(
        oh, tbl_ref[...], (((0,), (0,)), ((), ())),
        preferred_element_type=jnp.float32,
    )
    out_ref[...] = tok + emb


def kernel(tokens, scale_ids, frame_ids, codebook, W_proj, scale_table, frame_table):
    L = tokens.shape[0]
    D = codebook.shape[1]
    NE = W_proj.shape[0]
    n_scales = scale_table.shape[0]

    vec2 = _sc_gather_packed(codebook, tokens.astype(jnp.int32), 80, 28)

    W2 = (
        jnp.zeros((2, 2 * D, NE), jnp.float32)
        .at[0, :D].set(W_proj.T)
        .at[1, D:].set(W_proj.T)
    )
    tbl = (
        jnp.zeros((32, NE), jnp.float32)
        .at[:n_scales].set(scale_table)
        .at[16:18].set(frame_table)
    )
    BR = 1120
    grid = L // BR
    sids3 = scale_ids.astype(jnp.int32).reshape(grid, 1, BR)
    fids3 = frame_ids.astype(jnp.int32).reshape(grid, 1, BR)

    out = pl.pallas_call(
        functools.partial(_tc_body, n_scales),
        grid=(grid,),
        in_specs=[
            pl.BlockSpec((BR, 2 * D), lambda i: (0, 0)),
            pl.BlockSpec((1, 2 * D, NE), lambda i: (i, 0, 0)),
            pl.BlockSpec((1, 1, BR), lambda i: (i, 0, 0)),
            pl.BlockSpec((1, 1, BR), lambda i: (i, 0, 0)),
            pl.BlockSpec((32, NE), lambda i: (0, 0)),
        ],
        out_specs=pl.BlockSpec((BR, NE), lambda i: (i, 0)),
        out_shape=jax.ShapeDtypeStruct((L, NE), jnp.float32),
    )(vec2, W2, sids3, fids3, tbl)
    return out
